# baseline (device time: 41057 ns/iter reference)
import jax
import jax.numpy as jnp
from jax import lax
from jax.experimental import pallas as pl
from jax.experimental.pallas import tpu as pltpu

N_DEV = 16
N_TOK = 512
D_IN = 256
D_OUT = 512
N_EXP = 64
EXP_PER_DEV = 4
CAP = 6
SLOTS_PER_EXP = 8
SLOTS_PER_DEV = EXP_PER_DEV * SLOTS_PER_EXP
N_SLOTS = N_DEV * SLOTS_PER_DEV


def kernel(x, router_W, route_idx, expert_W):
    del router_W
    route_col = route_idx.astype(jnp.int32)
    route_row = route_col.reshape(1, N_TOK)

    def body(x_ref, rc_ref, rr_ref, ew_ref, out_ref,
             y_ref, comm_ref, send_sems, recv_sems):
        my = lax.axis_index("i")
        left = lax.rem(my + N_DEV - 1, N_DEV)
        right = lax.rem(my + 1, N_DEV)

        barrier_sem = pltpu.get_barrier_semaphore()
        for nbr in (left, right):
            pl.semaphore_signal(barrier_sem, inc=1, device_id=(nbr,),
                                device_id_type=pl.DeviceIdType.MESH)
        pl.semaphore_wait(barrier_sem, 2)

        rc = rc_ref[:, :]
        rr = rr_ref[:, :]

        a0 = lax.broadcasted_iota(jnp.int32, (N_TOK, N_TOK), 0)
        a1 = lax.broadcasted_iota(jnp.int32, (N_TOK, N_TOK), 1)

        ei = lax.broadcasted_iota(jnp.int32, (N_EXP, N_TOK), 0)
        Ot = (rr == ei).astype(jnp.bfloat16)
        UT = (a0 <= a1).astype(jnp.bfloat16)
        Ct = jax.lax.dot_general(Ot, UT, (((1,), (0,)), ((), ())),
                                 preferred_element_type=jnp.float32)
        cnt_row = jnp.sum(Ot.astype(jnp.float32) * Ct, axis=0,
                          keepdims=True).astype(jnp.int32)

        Oc = (rc == lax.broadcasted_iota(jnp.int32, (N_TOK, N_EXP), 1)
              ).astype(jnp.bfloat16)
        L = (a1 <= a0).astype(jnp.bfloat16)
        Cc = jax.lax.dot_general(L, Oc, (((1,), (0,)), ((), ())),
                                 preferred_element_type=jnp.float32)
        cnt_col = jnp.sum(Oc.astype(jnp.float32) * Cc, axis=1,
                          keepdims=True).astype(jnp.int32)

        si = lax.broadcasted_iota(jnp.int32, (SLOTS_PER_DEV, N_TOK), 0)
        Mt = ((rr == my * EXP_PER_DEV + si // SLOTS_PER_EXP)
              & (cnt_row == si % SLOTS_PER_EXP + 1)
              & (si % SLOTS_PER_EXP < CAP)).astype(jnp.bfloat16)

        x_sel = jnp.dot(Mt, x_ref[:, :].astype(jnp.bfloat16),
                        preferred_element_type=jnp.float32
                        ).astype(jnp.bfloat16)

        for j in range(EXP_PER_DEV):
            w = ew_ref[j, :, :].astype(jnp.bfloat16)
            yj = jnp.dot(x_sel[j * SLOTS_PER_EXP:(j + 1) * SLOTS_PER_EXP, :],
                         w, preferred_element_type=jnp.float32)
            comm_ref[0, j * SLOTS_PER_EXP:(j + 1) * SLOTS_PER_EXP, :] = (
                yj.astype(jnp.bfloat16))

        y_ref[pl.ds(my * SLOTS_PER_DEV, SLOTS_PER_DEV), :] = comm_ref[0, :, :]

        for h in range(N_DEV - 1):
            send_slot = h % 2
            recv_slot = (h + 1) % 2
            rdma = pltpu.make_async_remote_copy(
                src_ref=comm_ref.at[send_slot],
                dst_ref=comm_ref.at[recv_slot],
                send_sem=send_sems.at[send_slot],
                recv_sem=recv_sems.at[recv_slot],
                device_id=(right,),
                device_id_type=pl.DeviceIdType.MESH,
            )
            rdma.start()
            rdma.wait()
            origin = lax.rem(my + N_DEV - (h + 1), N_DEV)
            y_ref[pl.ds(origin * SLOTS_PER_DEV, SLOTS_PER_DEV), :] = (
                comm_ref[recv_slot, :, :])

        sg = lax.broadcasted_iota(jnp.int32, (N_TOK, N_SLOTS), 1)
        S = ((rc == sg // SLOTS_PER_EXP)
             & (cnt_col == sg % SLOTS_PER_EXP + 1)
             & (sg % SLOTS_PER_EXP < CAP)).astype(jnp.bfloat16)
        out_ref[:, :] = jnp.dot(S, y_ref[:, :],
                                preferred_element_type=jnp.float32)

    return pl.pallas_call(
        body,
        out_shape=jax.ShapeDtypeStruct((N_TOK, D_OUT), jnp.float32),
        in_specs=[
            pl.BlockSpec(memory_space=pltpu.VMEM),
            pl.BlockSpec(memory_space=pltpu.VMEM),
            pl.BlockSpec(memory_space=pltpu.VMEM),
            pl.BlockSpec(memory_space=pltpu.VMEM),
        ],
        out_specs=pl.BlockSpec(memory_space=pltpu.VMEM),
        scratch_shapes=[
            pltpu.VMEM((N_SLOTS, D_OUT), jnp.bfloat16),
            pltpu.VMEM((2, SLOTS_PER_DEV, D_OUT), jnp.bfloat16),
            pltpu.SemaphoreType.DMA((2,)),
            pltpu.SemaphoreType.DMA((2,)),
        ],
        compiler_params=pltpu.CompilerParams(collective_id=0),
    )(x, route_col, route_row, expert_W)


# device time: 24103 ns/iter; 1.7034x vs baseline; 1.7034x over previous
import functools

import jax
import jax.numpy as jnp
from jax import lax
from jax.experimental import pallas as pl
from jax.experimental.pallas import tpu as pltpu

N_DEV = 16
N_TOK = 512
D_IN = 256
D_OUT = 512
N_EXP = 64
EXP_PER_DEV = 4
CAP = 6
SLOTS_PER_EXP = 8
SLOTS_PER_DEV = EXP_PER_DEV * SLOTS_PER_EXP
N_SLOTS = N_DEV * SLOTS_PER_DEV


def kernel(x, router_W, route_idx, expert_W):
    del router_W
    route_col = route_idx.astype(jnp.int32)
    route_row = route_col.reshape(1, N_TOK)

    def body(x_ref, rc_ref, rr_ref, ew_ref, out_ref,
             y_ref, send_sems, recv_sems):
        my = lax.axis_index("i")

        rc = rc_ref[:, :]
        rr = rr_ref[:, :]

        a0 = lax.broadcasted_iota(jnp.int32, (N_TOK, N_TOK), 0)
        a1 = lax.broadcasted_iota(jnp.int32, (N_TOK, N_TOK), 1)

        ei = lax.broadcasted_iota(jnp.int32, (N_EXP, N_TOK), 0)
        Ot = (rr == ei).astype(jnp.bfloat16)
        UT = (a0 <= a1).astype(jnp.bfloat16)
        Ct = jax.lax.dot_general(Ot, UT, (((1,), (0,)), ((), ())),
                                 preferred_element_type=jnp.float32)
        cnt_row = jnp.sum(Ot.astype(jnp.float32) * Ct, axis=0,
                          keepdims=True).astype(jnp.int32)

        si = lax.broadcasted_iota(jnp.int32, (SLOTS_PER_DEV, N_TOK), 0)
        Mt = ((rr == my * EXP_PER_DEV + si // SLOTS_PER_EXP)
              & (cnt_row == si % SLOTS_PER_EXP + 1)
              & (si % SLOTS_PER_EXP < CAP)).astype(jnp.bfloat16)

        x_sel = jnp.dot(Mt, x_ref[:, :].astype(jnp.bfloat16),
                        preferred_element_type=jnp.float32
                        ).astype(jnp.bfloat16)

        for j in range(EXP_PER_DEV):
            w = ew_ref[j, :, :].astype(jnp.bfloat16)
            yj = jnp.dot(x_sel[j * SLOTS_PER_EXP:(j + 1) * SLOTS_PER_EXP, :],
                         w, preferred_element_type=jnp.float32)
            y_ref[pl.ds(my * SLOTS_PER_DEV + j * SLOTS_PER_EXP,
                        SLOTS_PER_EXP), :] = yj.astype(jnp.bfloat16)

        barrier_sem = pltpu.get_barrier_semaphore()
        for t in range(1, N_DEV):
            other = lax.rem(my + t, N_DEV)
            pl.semaphore_signal(barrier_sem, inc=1, device_id=(other,),
                                device_id_type=pl.DeviceIdType.MESH)
        pl.semaphore_wait(barrier_sem, N_DEV - 1)

        sends = []
        for t in range(1, N_DEV):
            target = lax.rem(my + t, N_DEV)
            rdma = pltpu.make_async_remote_copy(
                src_ref=y_ref.at[pl.ds(my * SLOTS_PER_DEV, SLOTS_PER_DEV)],
                dst_ref=y_ref.at[pl.ds(my * SLOTS_PER_DEV, SLOTS_PER_DEV)],
                send_sem=send_sems.at[t],
                recv_sem=recv_sems.at[my],
                device_id=(target,),
                device_id_type=pl.DeviceIdType.MESH,
            )
            rdma.start()
            sends.append(rdma)

        Oc = (rc == lax.broadcasted_iota(jnp.int32, (N_TOK, N_EXP), 1)
              ).astype(jnp.bfloat16)
        L = (a1 <= a0).astype(jnp.bfloat16)
        Cc = jax.lax.dot_general(L, Oc, (((1,), (0,)), ((), ())),
                                 preferred_element_type=jnp.float32)
        cnt_col = jnp.sum(Oc.astype(jnp.float32) * Cc, axis=1,
                          keepdims=True).astype(jnp.int32)

        sg = lax.broadcasted_iota(jnp.int32, (N_TOK, N_SLOTS), 1)
        S = ((rc == sg // SLOTS_PER_EXP)
             & (cnt_col == sg % SLOTS_PER_EXP + 1)
             & (sg % SLOTS_PER_EXP < CAP)).astype(jnp.bfloat16)

        for t in range(1, N_DEV):
            origin = lax.rem(my + t, N_DEV)
            recv = pltpu.make_async_remote_copy(
                src_ref=y_ref.at[pl.ds(origin * SLOTS_PER_DEV,
                                       SLOTS_PER_DEV)],
                dst_ref=y_ref.at[pl.ds(origin * SLOTS_PER_DEV,
                                       SLOTS_PER_DEV)],
                send_sem=send_sems.at[t],
                recv_sem=recv_sems.at[origin],
                device_id=(origin,),
                device_id_type=pl.DeviceIdType.MESH,
            )
            recv.wait_recv()

        out_ref[:, :] = jnp.dot(S, y_ref[:, :],
                                preferred_element_type=jnp.float32)

        for rdma in sends:
            rdma.wait_send()

        @functools.partial(pl.run_scoped,
                           second_barrier=pltpu.SemaphoreType.REGULAR)
        def _(second_barrier):
            for t in range(1, N_DEV):
                other = lax.rem(my + t, N_DEV)
                pl.semaphore_signal(second_barrier, inc=1,
                                    device_id=(other,),
                                    device_id_type=pl.DeviceIdType.MESH)
            pl.semaphore_wait(second_barrier, N_DEV - 1)

    return pl.pallas_call(
        body,
        out_shape=jax.ShapeDtypeStruct((N_TOK, D_OUT), jnp.float32),
        in_specs=[
            pl.BlockSpec(memory_space=pltpu.VMEM),
            pl.BlockSpec(memory_space=pltpu.VMEM),
            pl.BlockSpec(memory_space=pltpu.VMEM),
            pl.BlockSpec(memory_space=pltpu.VMEM),
        ],
        out_specs=pl.BlockSpec(memory_space=pltpu.VMEM),
        scratch_shapes=[
            pltpu.VMEM((N_SLOTS, D_OUT), jnp.bfloat16),
            pltpu.SemaphoreType.DMA((N_DEV,)),
            pltpu.SemaphoreType.DMA((N_DEV,)),
        ],
        compiler_params=pltpu.CompilerParams(collective_id=0),
    )(x, route_col, route_row, expert_W)


# device time: 18211 ns/iter; 2.2545x vs baseline; 1.3235x over previous
import jax
import jax.numpy as jnp
from jax import lax
from jax.experimental import pallas as pl
from jax.experimental.pallas import tpu as pltpu

N_DEV = 16
N_TOK = 512
D_IN = 256
D_OUT = 512
N_EXP = 64
EXP_PER_DEV = 4
CAP = 6
SLOTS_PER_EXP = 8
SLOTS_PER_DEV = EXP_PER_DEV * SLOTS_PER_EXP
N_SLOTS = N_DEV * SLOTS_PER_DEV


def kernel(x, router_W, route_idx, expert_W):
    del router_W
    route_col = route_idx.astype(jnp.int32)
    route_row = route_col.reshape(1, N_TOK)

    def body(x_ref, rc_ref, rr_ref, ew_ref, out_ref,
             y_ref, send_sems, recv_sems):
        my = lax.axis_index("i")

        rc = rc_ref[:, :]
        rr = rr_ref[:, :]

        a0 = lax.broadcasted_iota(jnp.int32, (N_TOK, N_TOK), 0)
        a1 = lax.broadcasted_iota(jnp.int32, (N_TOK, N_TOK), 1)

        ei = lax.broadcasted_iota(jnp.int32, (N_EXP, N_TOK), 0)
        Ot = (rr == ei).astype(jnp.bfloat16)
        UT = (a0 <= a1).astype(jnp.bfloat16)
        Ct = jax.lax.dot_general(Ot, UT, (((1,), (0,)), ((), ())),
                                 preferred_element_type=jnp.float32)
        cnt_row = jnp.sum(Ot.astype(jnp.float32) * Ct, axis=0,
                          keepdims=True).astype(jnp.int32)

        si = lax.broadcasted_iota(jnp.int32, (SLOTS_PER_DEV, N_TOK), 0)
        Mt = ((rr == my * EXP_PER_DEV + si // SLOTS_PER_EXP)
              & (cnt_row == si % SLOTS_PER_EXP + 1)
              & (si % SLOTS_PER_EXP < CAP)).astype(jnp.bfloat16)

        x_sel = jnp.dot(Mt, x_ref[:, :].astype(jnp.bfloat16),
                        preferred_element_type=jnp.float32
                        ).astype(jnp.bfloat16)

        for j in range(EXP_PER_DEV):
            w = ew_ref[j, :, :].astype(jnp.bfloat16)
            yj = jnp.dot(x_sel[j * SLOTS_PER_EXP:(j + 1) * SLOTS_PER_EXP, :],
                         w, preferred_element_type=jnp.float32)
            y_ref[pl.ds(my * SLOTS_PER_DEV + j * SLOTS_PER_EXP,
                        SLOTS_PER_EXP), :] = yj.astype(jnp.bfloat16)

        barrier_sem = pltpu.get_barrier_semaphore()
        for t in range(1, N_DEV):
            other = lax.rem(my + t, N_DEV)
            pl.semaphore_signal(barrier_sem, inc=1, device_id=(other,),
                                device_id_type=pl.DeviceIdType.MESH)
        pl.semaphore_wait(barrier_sem, N_DEV - 1)

        sends = []
        for t in range(1, N_DEV):
            target = lax.rem(my + t, N_DEV)
            rdma = pltpu.make_async_remote_copy(
                src_ref=y_ref.at[pl.ds(my * SLOTS_PER_DEV, SLOTS_PER_DEV)],
                dst_ref=y_ref.at[pl.ds(my * SLOTS_PER_DEV, SLOTS_PER_DEV)],
                send_sem=send_sems.at[t],
                recv_sem=recv_sems.at[my],
                device_id=(target,),
                device_id_type=pl.DeviceIdType.MESH,
            )
            rdma.start()
            sends.append(rdma)

        Oc = (rc == lax.broadcasted_iota(jnp.int32, (N_TOK, N_EXP), 1)
              ).astype(jnp.bfloat16)
        L = (a1 <= a0).astype(jnp.bfloat16)
        Cc = jax.lax.dot_general(L, Oc, (((1,), (0,)), ((), ())),
                                 preferred_element_type=jnp.float32)
        cnt_col = jnp.sum(Oc.astype(jnp.float32) * Cc, axis=1,
                          keepdims=True).astype(jnp.int32)

        sg = lax.broadcasted_iota(jnp.int32, (N_TOK, N_SLOTS), 1)
        S = ((rc == sg // SLOTS_PER_EXP)
             & (cnt_col == sg % SLOTS_PER_EXP + 1)
             & (sg % SLOTS_PER_EXP < CAP)).astype(jnp.bfloat16)

        for t in range(1, N_DEV):
            origin = lax.rem(my + t, N_DEV)
            recv = pltpu.make_async_remote_copy(
                src_ref=y_ref.at[pl.ds(origin * SLOTS_PER_DEV,
                                       SLOTS_PER_DEV)],
                dst_ref=y_ref.at[pl.ds(origin * SLOTS_PER_DEV,
                                       SLOTS_PER_DEV)],
                send_sem=send_sems.at[t],
                recv_sem=recv_sems.at[origin],
                device_id=(origin,),
                device_id_type=pl.DeviceIdType.MESH,
            )
            recv.wait_recv()

        out_ref[:, :] = jnp.dot(S, y_ref[:, :],
                                preferred_element_type=jnp.float32)

        for rdma in sends:
            rdma.wait_send()


    return pl.pallas_call(
        body,
        out_shape=jax.ShapeDtypeStruct((N_TOK, D_OUT), jnp.float32),
        in_specs=[
            pl.BlockSpec(memory_space=pltpu.VMEM),
            pl.BlockSpec(memory_space=pltpu.VMEM),
            pl.BlockSpec(memory_space=pltpu.VMEM),
            pl.BlockSpec(memory_space=pltpu.VMEM),
        ],
        out_specs=pl.BlockSpec(memory_space=pltpu.VMEM),
        scratch_shapes=[
            pltpu.VMEM((N_SLOTS, D_OUT), jnp.bfloat16),
            pltpu.SemaphoreType.DMA((N_DEV,)),
            pltpu.SemaphoreType.DMA((N_DEV,)),
        ],
        compiler_params=pltpu.CompilerParams(collective_id=0),
    )(x, route_col, route_row, expert_W)


# device time: 15147 ns/iter; 2.7106x vs baseline; 1.2023x over previous
import jax
import jax.numpy as jnp
from jax import lax
from jax.experimental import pallas as pl
from jax.experimental.pallas import tpu as pltpu

N_DEV = 16
N_TOK = 512
D_IN = 256
D_OUT = 512
N_EXP = 64
EXP_PER_DEV = 4
CAP = 6
SLOTS_PER_EXP = CAP
SLOTS_PER_DEV = EXP_PER_DEV * SLOTS_PER_EXP
N_SLOTS = N_DEV * SLOTS_PER_DEV


def kernel(x, router_W, route_idx, expert_W):
    del router_W
    route_col = route_idx.astype(jnp.int32)

    def body(x_ref, rc_ref, ew_ref, out_ref, y_ref, send_sems, recv_sems):
        my = lax.axis_index("i")

        barrier_sem = pltpu.get_barrier_semaphore()
        for t in range(1, N_DEV):
            other = lax.rem(my + t, N_DEV)
            pl.semaphore_signal(barrier_sem, inc=1, device_id=(other,),
                                device_id_type=pl.DeviceIdType.MESH)

        rc = rc_ref[:, :]

        Oc = (rc == lax.broadcasted_iota(jnp.int32, (N_TOK, N_EXP), 1)
              ).astype(jnp.bfloat16)
        a0 = lax.broadcasted_iota(jnp.int32, (N_TOK, N_TOK), 0)
        a1 = lax.broadcasted_iota(jnp.int32, (N_TOK, N_TOK), 1)
        L = (a1 <= a0).astype(jnp.bfloat16)
        Cc = jax.lax.dot_general(L, Oc, (((1,), (0,)), ((), ())),
                                 preferred_element_type=jnp.float32)
        cnt_col = jnp.sum(Oc.astype(jnp.float32) * Cc, axis=1,
                          keepdims=True).astype(jnp.int32)

        si = lax.broadcasted_iota(jnp.int32, (N_TOK, SLOTS_PER_DEV), 1)
        Mc = ((rc == my * EXP_PER_DEV + si // SLOTS_PER_EXP)
              & (cnt_col == si % SLOTS_PER_EXP + 1)).astype(jnp.bfloat16)

        x_sel = jax.lax.dot_general(
            Mc, x_ref[:, :].astype(jnp.bfloat16),
            (((0,), (0,)), ((), ())),
            preferred_element_type=jnp.float32).astype(jnp.bfloat16)

        ys = []
        for j in range(EXP_PER_DEV):
            w = ew_ref[j, :, :].astype(jnp.bfloat16)
            ys.append(jnp.dot(
                x_sel[j * SLOTS_PER_EXP:(j + 1) * SLOTS_PER_EXP, :],
                w, preferred_element_type=jnp.float32))
        y_ref[pl.ds(my * SLOTS_PER_DEV, SLOTS_PER_DEV), :] = (
            jnp.concatenate(ys, axis=0).astype(jnp.bfloat16))

        pl.semaphore_wait(barrier_sem, N_DEV - 1)

        sends = []
        for t in range(1, N_DEV):
            target = lax.rem(my + t, N_DEV)
            rdma = pltpu.make_async_remote_copy(
                src_ref=y_ref.at[pl.ds(my * SLOTS_PER_DEV, SLOTS_PER_DEV)],
                dst_ref=y_ref.at[pl.ds(my * SLOTS_PER_DEV, SLOTS_PER_DEV)],
                send_sem=send_sems.at[t],
                recv_sem=recv_sems.at[my],
                device_id=(target,),
                device_id_type=pl.DeviceIdType.MESH,
            )
            rdma.start()
            sends.append(rdma)

        sg = lax.broadcasted_iota(jnp.int32, (N_TOK, N_SLOTS), 1)
        S = ((rc == sg // SLOTS_PER_EXP)
             & (cnt_col == sg % SLOTS_PER_EXP + 1)).astype(jnp.bfloat16)

        for t in range(1, N_DEV):
            origin = lax.rem(my + t, N_DEV)
            recv = pltpu.make_async_remote_copy(
                src_ref=y_ref.at[pl.ds(origin * SLOTS_PER_DEV,
                                       SLOTS_PER_DEV)],
                dst_ref=y_ref.at[pl.ds(origin * SLOTS_PER_DEV,
                                       SLOTS_PER_DEV)],
                send_sem=send_sems.at[t],
                recv_sem=recv_sems.at[origin],
                device_id=(origin,),
                device_id_type=pl.DeviceIdType.MESH,
            )
            recv.wait_recv()

        out_ref[:, :] = jnp.dot(S, y_ref[:, :],
                                preferred_element_type=jnp.float32
                                ).astype(jnp.bfloat16)

        for rdma in sends:
            rdma.wait_send()

    return pl.pallas_call(
        body,
        out_shape=jax.ShapeDtypeStruct((N_TOK, D_OUT), jnp.bfloat16),
        in_specs=[
            pl.BlockSpec(memory_space=pltpu.VMEM),
            pl.BlockSpec(memory_space=pltpu.VMEM),
            pl.BlockSpec(memory_space=pltpu.VMEM),
        ],
        out_specs=pl.BlockSpec(memory_space=pltpu.VMEM),
        scratch_shapes=[
            pltpu.VMEM((N_SLOTS, D_OUT), jnp.bfloat16),
            pltpu.SemaphoreType.DMA((N_DEV,)),
            pltpu.SemaphoreType.DMA((N_DEV,)),
        ],
        compiler_params=pltpu.CompilerParams(collective_id=0),
    )(x, route_col, expert_W)
